# baseline (speedup 1.0000x reference)
import functools

import jax
import jax.numpy as jnp
from jax import lax
from jax.experimental import pallas as pl
from jax.experimental.pallas import tpu as pltpu
from jax.experimental.pallas import tpu_sc as plsc

NC = 2
NS = 16
LANES = 16
CHUNK = 128
ROW_BLK = 400
DEG_W = 128
UNROLL = 8


def _mesh():
    return plsc.VectorSubcoreMesh(
        core_axis_name="c", subcore_axis_name="s", num_cores=NC, num_subcores=NS
    )


def _deg_body(k_per_tile, stripe, hn, npass, width,
              dst2d, ones, zeros, out, dstv, dstl, onesv, acc):
    c = lax.axis_index("c")
    s = lax.axis_index("s")
    base = (c * NS + s) * k_per_tile
    pltpu.sync_copy(dst2d.at[pl.ds(base, k_per_tile)], dstv)
    pltpu.sync_copy(ones, onesv)

    for p in range(npass):
        lo = jnp.full((LANES,), p * hn, jnp.int32)
        dummy = jnp.full((LANES,), hn, jnp.int32)

        def remap(r, carry):
            for kk in range(CHUNK // LANES):
                sl = pl.ds(kk * LANES, LANES)
                local = dstv[r, sl] - lo
                oob = (local < 0) | (local >= hn)
                dstl[r, sl] = jnp.where(oob, dummy, local)
            return carry

        lax.fori_loop(0, k_per_tile, remap, 0)
        pltpu.sync_copy(zeros, acc.at[pl.ds(s * stripe, stripe)])
        plsc.subcore_barrier()

        def step(j, carry):
            pltpu.sync_copy(onesv, acc.at[dstl.at[j]], add=True)
            return carry

        lax.fori_loop(0, k_per_tile, step, 0)
        plsc.subcore_barrier()
        pltpu.sync_copy(acc.at[pl.ds(s * stripe, stripe)],
                        out.at[c, pl.ds(p * hn + s * stripe, stripe)])


@functools.lru_cache(maxsize=None)
def _make_deg_kernel(nchunks, rpad, width, hn, npass):
    stripe = hn // NS
    k = nchunks // (NC * NS)
    return pl.kernel(
        functools.partial(_deg_body, k, stripe, hn, npass, width),
        out_type=jax.ShapeDtypeStruct((NC, rpad, width), jnp.float32),
        mesh=_mesh(),
        scratch_types=[
            pltpu.VMEM((k, CHUNK), jnp.int32),
            pltpu.VMEM((k, CHUNK), jnp.int32),
            pltpu.VMEM((CHUNK, width), jnp.float32),
            pltpu.VMEM_SHARED((hn + 8, width), jnp.float32),
        ],
    )


def _scatter_body(feature_split, n, k_per_tile, stripe, hn, npass,
                  src2d, dst2d, table, zeros, out,
                  srcv, dstv, dstl, rows0, rows1, acc, sem0, sem1):
    c = lax.axis_index("c")
    s = lax.axis_index("s")
    if feature_split:
        base = s * k_per_tile
    else:
        base = (c * NS + s) * k_per_tile
    pltpu.sync_copy(src2d.at[pl.ds(base, k_per_tile)], srcv)
    pltpu.sync_copy(dst2d.at[pl.ds(base, k_per_tile)], dstv)
    if feature_split:
        off = jnp.full((LANES,), c * n, jnp.int32)

        def adj(r, carry):
            for kk in range(CHUNK // LANES):
                sl = pl.ds(kk * LANES, LANES)
                srcv[r, sl] = srcv[r, sl] + off
            return carry

        lax.fori_loop(0, k_per_tile, adj, 0)

    for p in range(npass):
        lo = jnp.full((LANES,), p * hn, jnp.int32)
        dummy = jnp.full((LANES,), hn, jnp.int32)

        def remap(r, carry):
            for kk in range(CHUNK // LANES):
                sl = pl.ds(kk * LANES, LANES)
                local = dstv[r, sl] - lo
                oob = (local < 0) | (local >= hn)
                dstl[r, sl] = jnp.where(oob, dummy, local)
            return carry

        lax.fori_loop(0, k_per_tile, remap, 0)
        pltpu.sync_copy(zeros, acc.at[pl.ds(s * stripe, stripe)])
        plsc.subcore_barrier()

        bufs = (rows0, rows1)
        sems = (sem0, sem1)

        def block(blk, carry):
            base = blk * UNROLL
            ds = [None] * UNROLL
            ds[0] = pltpu.async_copy(
                table.at[srcv.at[base]], bufs[0], sems[0])
            for b in range(UNROLL):
                if b + 1 < UNROLL:
                    ds[b + 1] = pltpu.async_copy(
                        table.at[srcv.at[base + b + 1]],
                        bufs[(b + 1) % 2], sems[(b + 1) % 2])
                ds[b].wait()
                pltpu.sync_copy(bufs[b % 2], acc.at[dstl.at[base + b]],
                                add=True)
            return carry

        lax.fori_loop(0, k_per_tile // UNROLL, block, 0)
        plsc.subcore_barrier()
        pltpu.sync_copy(acc.at[pl.ds(s * stripe, stripe)],
                        out.at[c, pl.ds(p * hn + s * stripe, stripe)])


@functools.lru_cache(maxsize=None)
def _make_scatter_kernel(feature_split, n, nchunks, rpad, width, hn, npass):
    stripe = hn // NS
    k = nchunks // NS if feature_split else nchunks // (NC * NS)
    return pl.kernel(
        functools.partial(_scatter_body, feature_split, n, k, stripe, hn,
                          npass),
        out_type=jax.ShapeDtypeStruct((NC, rpad, width), jnp.float32),
        mesh=_mesh(),
        scratch_types=[
            pltpu.VMEM((k, CHUNK), jnp.int32),
            pltpu.VMEM((k, CHUNK), jnp.int32),
            pltpu.VMEM((k, CHUNK), jnp.int32),
            pltpu.VMEM((CHUNK, width), jnp.float32),
            pltpu.VMEM((CHUNK, width), jnp.float32),
            pltpu.VMEM_SHARED((hn + 8, width), jnp.float32),
            pltpu.SemaphoreType.DMA,
            pltpu.SemaphoreType.DMA,
        ],
    )


def _dis(degp_ref):
    deg = degp_ref[0, :, 0:1] + degp_ref[1, :, 0:1] + 1.0
    return lax.rsqrt(deg)


def _tc_first_body(x_ref, degp_ref, w_ref, hp_ref):
    dis = _dis(degp_ref)
    h = jnp.dot(x_ref[...], w_ref[...], preferred_element_type=jnp.float32)
    hp = h * dis
    half = hp.shape[1] // 2
    hp_ref[0] = hp[:, :half]
    hp_ref[1] = hp[:, half:]


def _tc_mid_body(split_out, degp_ref, acc_ref, hpp_ref, b_ref, w_ref, hp_ref):
    dis = _dis(degp_ref)
    cat = jnp.concatenate([acc_ref[0] + hpp_ref[0], acc_ref[1] + hpp_ref[1]],
                          axis=1)
    x = jnp.maximum(cat * dis + b_ref[...], 0.0)
    hp = jnp.dot(x, w_ref[...], preferred_element_type=jnp.float32) * dis
    if split_out:
        half = hp.shape[1] // 2
        hp_ref[0] = hp[:, :half]
        hp_ref[1] = hp[:, half:]
    else:
        hp_ref[...] = hp


def _tc_last_body(degp_ref, accp_ref, hp_ref, b_ref, out_ref):
    dis = _dis(degp_ref)
    t = (accp_ref[0] + accp_ref[1] + hp_ref[...]) * dis + b_ref[...]
    out_ref[0] = jnp.maximum(t, 0.0)


def _tc_first(x, degp, w):
    n, d_in = x.shape
    d_out = w.shape[1]
    grid = n // ROW_BLK
    return pl.pallas_call(
        _tc_first_body,
        grid=(grid,),
        in_specs=[
            pl.BlockSpec((ROW_BLK, d_in), lambda i: (i, 0)),
            pl.BlockSpec((NC, ROW_BLK, DEG_W), lambda i: (0, i, 0)),
            pl.BlockSpec((d_in, d_out), lambda i: (0, 0)),
        ],
        out_specs=pl.BlockSpec((NC, ROW_BLK, d_out // 2), lambda i: (0, i, 0)),
        out_shape=jax.ShapeDtypeStruct((NC, n, d_out // 2), jnp.float32),
    )(x, degp, w)


def _tc_mid(degp, acc, hpp, b, w, split_out):
    n = acc.shape[1]
    d_in = 2 * acc.shape[2]
    d_out = w.shape[1]
    grid = n // ROW_BLK
    if split_out:
        out_spec = pl.BlockSpec((NC, ROW_BLK, d_out // 2), lambda i: (0, i, 0))
        out_shape = jax.ShapeDtypeStruct((NC, n, d_out // 2), jnp.float32)
    else:
        out_spec = pl.BlockSpec((ROW_BLK, d_out), lambda i: (i, 0))
        out_shape = jax.ShapeDtypeStruct((n, d_out), jnp.float32)
    return pl.pallas_call(
        functools.partial(_tc_mid_body, split_out),
        grid=(grid,),
        in_specs=[
            pl.BlockSpec((NC, ROW_BLK, DEG_W), lambda i: (0, i, 0)),
            pl.BlockSpec((NC, ROW_BLK, d_in // 2), lambda i: (0, i, 0)),
            pl.BlockSpec((NC, ROW_BLK, d_in // 2), lambda i: (0, i, 0)),
            pl.BlockSpec((1, d_in), lambda i: (0, 0)),
            pl.BlockSpec((d_in, d_out), lambda i: (0, 0)),
        ],
        out_specs=out_spec,
        out_shape=out_shape,
    )(degp, acc, hpp, b, w)


def _tc_last(degp, accp, hp, b):
    n, d = hp.shape
    grid = n // ROW_BLK
    return pl.pallas_call(
        _tc_last_body,
        grid=(grid,),
        in_specs=[
            pl.BlockSpec((NC, ROW_BLK, DEG_W), lambda i: (0, i, 0)),
            pl.BlockSpec((NC, ROW_BLK, d), lambda i: (0, i, 0)),
            pl.BlockSpec((ROW_BLK, d), lambda i: (i, 0)),
            pl.BlockSpec((1, d), lambda i: (0, 0)),
        ],
        out_specs=pl.BlockSpec((1, ROW_BLK, d), lambda i: (0, i, 0)),
        out_shape=jax.ShapeDtypeStruct((1, n, d), jnp.float32),
    )(degp, accp, hp, b)


def kernel(mesh_node_features, edge_index, W1, b1, W2, b2, W3, b3):
    x = mesh_node_features[0]
    n = x.shape[0]
    e = edge_index.shape[1]
    h_dim = W1.shape[1]

    npass = 3
    hn = -(-(-(-(n + 1) // npass)) // (NS * 8)) * (NS * 8)
    rpad = npass * hn
    nchunks = -(-e // CHUNK)
    nchunks = -(-nchunks // (NC * NS * 8)) * (NC * NS * 8)
    pad_e = nchunks * CHUNK - e

    src = jnp.concatenate([edge_index[0], jnp.zeros((pad_e,), jnp.int32)])
    dst = jnp.concatenate([edge_index[1], jnp.full((pad_e,), n, jnp.int32)])
    src2d = src.reshape(nchunks, CHUNK)
    dst2d = dst.reshape(nchunks, CHUNK)

    zeros_h = jnp.zeros((hn // NS, h_dim // 2), jnp.float32)
    ones_deg = jnp.ones((CHUNK, DEG_W), jnp.float32)
    b1r = b1.reshape(1, -1)
    b2r = b2.reshape(1, -1)
    b3r = b3.reshape(1, -1)

    degp = _make_deg_kernel(nchunks, rpad, DEG_W, hn, npass)(
        dst2d, ones_deg, zeros_h)[:, :n, :]

    scat_f = _make_scatter_kernel(True, n, nchunks, rpad, h_dim // 2, hn,
                                  npass)
    scat_e = _make_scatter_kernel(False, n, nchunks, rpad, h_dim // 2, hn,
                                  npass)

    hp1 = _tc_first(x, degp, W1)
    acc1 = scat_f(src2d, dst2d, hp1.reshape(NC * n, -1), zeros_h)[:, :n, :]
    hp2 = _tc_mid(degp, acc1, hp1, b1r, W2, split_out=True)
    acc2 = scat_f(src2d, dst2d, hp2.reshape(NC * n, -1), zeros_h)[:, :n, :]
    hp3 = _tc_mid(degp, acc2, hp2, b2r, W3, split_out=False)
    acc3 = scat_e(src2d, dst2d, hp3, zeros_h)[:, :n, :]
    return _tc_last(degp, acc3, hp3, b3r)


# speedup vs baseline: 1.5862x; 1.5862x over previous
import functools

import jax
import jax.numpy as jnp
from jax import lax
from jax.experimental import pallas as pl
from jax.experimental.pallas import tpu as pltpu
from jax.experimental.pallas import tpu_sc as plsc

NC = 2
NS = 16
LANES = 16
CHUNK = 128
ROW_BLK = 400
DEG_W = 128
UNROLL = 8


def _mesh():
    return plsc.VectorSubcoreMesh(
        core_axis_name="c", subcore_axis_name="s", num_cores=NC, num_subcores=NS
    )


def _deg_body(k_per_tile, stripe, hn, npass, width,
              dst2d, ones, zeros, out, dstv, dstl, onesv, acc):
    c = lax.axis_index("c")
    s = lax.axis_index("s")
    base = (c * NS + s) * k_per_tile
    pltpu.sync_copy(dst2d.at[pl.ds(base, k_per_tile)], dstv)
    pltpu.sync_copy(ones, onesv)

    iota = lax.iota(jnp.int32, LANES)
    for p in range(npass):
        lo = jnp.full((LANES,), p * hn, jnp.int32)

        def remap(r, carry):
            for kk in range(CHUNK // LANES):
                sl = pl.ds(kk * LANES, LANES)
                local = dstv[r, sl] - lo
                oob = (local < 0) | (local >= hn)
                dummy = hn + kk * LANES + iota
                dstl[r, sl] = jnp.where(oob, dummy, local)
            return carry

        lax.fori_loop(0, k_per_tile, remap, 0)
        pltpu.sync_copy(zeros, acc.at[pl.ds(s * stripe, stripe)])
        plsc.subcore_barrier()

        def step(j, carry):
            pltpu.sync_copy(onesv, acc.at[dstl.at[j]], add=True)
            return carry

        lax.fori_loop(0, k_per_tile, step, 0)
        plsc.subcore_barrier()
        pltpu.sync_copy(acc.at[pl.ds(s * stripe, stripe)],
                        out.at[c, pl.ds(p * hn + s * stripe, stripe)])


@functools.lru_cache(maxsize=None)
def _make_deg_kernel(nchunks, rpad, width, hn, npass):
    stripe = hn // NS
    k = nchunks // (NC * NS)
    return pl.kernel(
        functools.partial(_deg_body, k, stripe, hn, npass, width),
        out_type=jax.ShapeDtypeStruct((NC, rpad, width), jnp.float32),
        mesh=_mesh(),
        scratch_types=[
            pltpu.VMEM((k, CHUNK), jnp.int32),
            pltpu.VMEM((k, CHUNK), jnp.int32),
            pltpu.VMEM((CHUNK, width), jnp.float32),
            pltpu.VMEM_SHARED((hn + CHUNK, width), jnp.float32),
        ],
    )


def _scatter_body(feature_split, n, k_per_tile, stripe, hn, npass,
                  src2d, dst2d, table, zeros, out,
                  srcv, dstv, dstl, rows0, rows1, acc, sem0, sem1):
    c = lax.axis_index("c")
    s = lax.axis_index("s")
    if feature_split:
        base = s * k_per_tile
    else:
        base = (c * NS + s) * k_per_tile
    pltpu.sync_copy(src2d.at[pl.ds(base, k_per_tile)], srcv)
    pltpu.sync_copy(dst2d.at[pl.ds(base, k_per_tile)], dstv)
    if feature_split:
        off = jnp.full((LANES,), c * n, jnp.int32)

        def adj(r, carry):
            for kk in range(CHUNK // LANES):
                sl = pl.ds(kk * LANES, LANES)
                srcv[r, sl] = srcv[r, sl] + off
            return carry

        lax.fori_loop(0, k_per_tile, adj, 0)

    iota = lax.iota(jnp.int32, LANES)
    for p in range(npass):
        lo = jnp.full((LANES,), p * hn, jnp.int32)

        def remap(r, carry):
            for kk in range(CHUNK // LANES):
                sl = pl.ds(kk * LANES, LANES)
                local = dstv[r, sl] - lo
                oob = (local < 0) | (local >= hn)
                dummy = hn + kk * LANES + iota
                dstl[r, sl] = jnp.where(oob, dummy, local)
            return carry

        lax.fori_loop(0, k_per_tile, remap, 0)
        pltpu.sync_copy(zeros, acc.at[pl.ds(s * stripe, stripe)])
        plsc.subcore_barrier()

        def step(j, carry):
            pltpu.async_copy(table.at[srcv.at[j]], rows0, sem0).wait()
            pltpu.sync_copy(rows0, acc.at[dstl.at[j]], add=True)
            return carry

        lax.fori_loop(0, k_per_tile, step, 0)
        plsc.subcore_barrier()
        pltpu.sync_copy(acc.at[pl.ds(s * stripe, stripe)],
                        out.at[c, pl.ds(p * hn + s * stripe, stripe)])


@functools.lru_cache(maxsize=None)
def _make_scatter_kernel(feature_split, n, nchunks, rpad, width, hn, npass):
    stripe = hn // NS
    k = nchunks // NS if feature_split else nchunks // (NC * NS)
    return pl.kernel(
        functools.partial(_scatter_body, feature_split, n, k, stripe, hn,
                          npass),
        out_type=jax.ShapeDtypeStruct((NC, rpad, width), jnp.float32),
        mesh=_mesh(),
        scratch_types=[
            pltpu.VMEM((k, CHUNK), jnp.int32),
            pltpu.VMEM((k, CHUNK), jnp.int32),
            pltpu.VMEM((k, CHUNK), jnp.int32),
            pltpu.VMEM((CHUNK, width), jnp.float32),
            pltpu.VMEM((CHUNK, width), jnp.float32),
            pltpu.VMEM_SHARED((hn + CHUNK, width), jnp.float32),
            pltpu.SemaphoreType.DMA,
            pltpu.SemaphoreType.DMA,
        ],
    )


def _dis(degp_ref):
    deg = degp_ref[0, :, 0:1] + degp_ref[1, :, 0:1] + 1.0
    return lax.rsqrt(deg)


def _tc_first_body(x_ref, degp_ref, w_ref, hp_ref):
    dis = _dis(degp_ref)
    h = jnp.dot(x_ref[...], w_ref[...], preferred_element_type=jnp.float32)
    hp = h * dis
    half = hp.shape[1] // 2
    hp_ref[0] = hp[:, :half]
    hp_ref[1] = hp[:, half:]


def _tc_mid_body(split_out, degp_ref, acc_ref, hpp_ref, b_ref, w_ref, hp_ref):
    dis = _dis(degp_ref)
    cat = jnp.concatenate([acc_ref[0] + hpp_ref[0], acc_ref[1] + hpp_ref[1]],
                          axis=1)
    x = jnp.maximum(cat * dis + b_ref[...], 0.0)
    hp = jnp.dot(x, w_ref[...], preferred_element_type=jnp.float32) * dis
    if split_out:
        half = hp.shape[1] // 2
        hp_ref[0] = hp[:, :half]
        hp_ref[1] = hp[:, half:]
    else:
        hp_ref[...] = hp


def _tc_last_body(degp_ref, accp_ref, hp_ref, b_ref, out_ref):
    dis = _dis(degp_ref)
    t = (accp_ref[0] + accp_ref[1] + hp_ref[...]) * dis + b_ref[...]
    out_ref[0] = jnp.maximum(t, 0.0)


def _tc_first(x, degp, w):
    n, d_in = x.shape
    d_out = w.shape[1]
    grid = n // ROW_BLK
    return pl.pallas_call(
        _tc_first_body,
        grid=(grid,),
        in_specs=[
            pl.BlockSpec((ROW_BLK, d_in), lambda i: (i, 0)),
            pl.BlockSpec((NC, ROW_BLK, DEG_W), lambda i: (0, i, 0)),
            pl.BlockSpec((d_in, d_out), lambda i: (0, 0)),
        ],
        out_specs=pl.BlockSpec((NC, ROW_BLK, d_out // 2), lambda i: (0, i, 0)),
        out_shape=jax.ShapeDtypeStruct((NC, n, d_out // 2), jnp.float32),
    )(x, degp, w)


def _tc_mid(degp, acc, hpp, b, w, split_out):
    n = acc.shape[1]
    d_in = 2 * acc.shape[2]
    d_out = w.shape[1]
    grid = n // ROW_BLK
    if split_out:
        out_spec = pl.BlockSpec((NC, ROW_BLK, d_out // 2), lambda i: (0, i, 0))
        out_shape = jax.ShapeDtypeStruct((NC, n, d_out // 2), jnp.float32)
    else:
        out_spec = pl.BlockSpec((ROW_BLK, d_out), lambda i: (i, 0))
        out_shape = jax.ShapeDtypeStruct((n, d_out), jnp.float32)
    return pl.pallas_call(
        functools.partial(_tc_mid_body, split_out),
        grid=(grid,),
        in_specs=[
            pl.BlockSpec((NC, ROW_BLK, DEG_W), lambda i: (0, i, 0)),
            pl.BlockSpec((NC, ROW_BLK, d_in // 2), lambda i: (0, i, 0)),
            pl.BlockSpec((NC, ROW_BLK, d_in // 2), lambda i: (0, i, 0)),
            pl.BlockSpec((1, d_in), lambda i: (0, 0)),
            pl.BlockSpec((d_in, d_out), lambda i: (0, 0)),
        ],
        out_specs=out_spec,
        out_shape=out_shape,
    )(degp, acc, hpp, b, w)


def _tc_last(degp, accp, hp, b):
    n, d = hp.shape
    grid = n // ROW_BLK
    return pl.pallas_call(
        _tc_last_body,
        grid=(grid,),
        in_specs=[
            pl.BlockSpec((NC, ROW_BLK, DEG_W), lambda i: (0, i, 0)),
            pl.BlockSpec((NC, ROW_BLK, d), lambda i: (0, i, 0)),
            pl.BlockSpec((ROW_BLK, d), lambda i: (i, 0)),
            pl.BlockSpec((1, d), lambda i: (0, 0)),
        ],
        out_specs=pl.BlockSpec((1, ROW_BLK, d), lambda i: (0, i, 0)),
        out_shape=jax.ShapeDtypeStruct((1, n, d), jnp.float32),
    )(degp, accp, hp, b)


def kernel(mesh_node_features, edge_index, W1, b1, W2, b2, W3, b3):
    x = mesh_node_features[0]
    n = x.shape[0]
    e = edge_index.shape[1]
    h_dim = W1.shape[1]

    npass = 2
    hn = -(-(-(-(n + 1) // npass)) // (NS * 8)) * (NS * 8)
    rpad = npass * hn
    nchunks = -(-e // CHUNK)
    nchunks = -(-nchunks // (NC * NS * 8)) * (NC * NS * 8)
    pad_e = nchunks * CHUNK - e

    src = jnp.concatenate([edge_index[0], jnp.zeros((pad_e,), jnp.int32)])
    dst = jnp.concatenate([edge_index[1], jnp.full((pad_e,), n, jnp.int32)])
    src2d = src.reshape(nchunks, CHUNK)
    dst2d = dst.reshape(nchunks, CHUNK)

    zeros_h = jnp.zeros((hn // NS, h_dim // 2), jnp.float32)
    ones_deg = jnp.ones((CHUNK, DEG_W), jnp.float32)
    b1r = b1.reshape(1, -1)
    b2r = b2.reshape(1, -1)
    b3r = b3.reshape(1, -1)

    degp = _make_deg_kernel(nchunks, rpad, DEG_W, hn, npass)(
        dst2d, ones_deg, zeros_h)[:, :n, :]

    scat_f = _make_scatter_kernel(True, n, nchunks, rpad, h_dim // 2, hn,
                                  npass)
    scat_e = _make_scatter_kernel(False, n, nchunks, rpad, h_dim // 2, hn,
                                  npass)

    hp1 = _tc_first(x, degp, W1)
    acc1 = scat_f(src2d, dst2d, hp1.reshape(NC * n, -1), zeros_h)[:, :n, :]
    hp2 = _tc_mid(degp, acc1, hp1, b1r, W2, split_out=True)
    acc2 = scat_f(src2d, dst2d, hp2.reshape(NC * n, -1), zeros_h)[:, :n, :]
    hp3 = _tc_mid(degp, acc2, hp2, b2r, W3, split_out=False)
    acc3 = scat_e(src2d, dst2d, hp3, zeros_h)[:, :n, :]
    return _tc_last(degp, acc3, hp3, b3r)


# speedup vs baseline: 1.5875x; 1.0008x over previous
import functools

import jax
import jax.numpy as jnp
from jax import lax
from jax.experimental import pallas as pl
from jax.experimental.pallas import tpu as pltpu
from jax.experimental.pallas import tpu_sc as plsc

NC = 2
NS = 16
LANES = 16
CHUNK = 128
ROW_BLK = 400
DEG_W = 128
UNROLL = 8


def _mesh():
    return plsc.VectorSubcoreMesh(
        core_axis_name="c", subcore_axis_name="s", num_cores=NC, num_subcores=NS
    )


def _deg_body(k_per_tile, stripe, hn, npass, width,
              dst2d, ones, zeros, out, dstv, dstl, onesv, acc):
    c = lax.axis_index("c")
    s = lax.axis_index("s")
    base = (c * NS + s) * k_per_tile
    pltpu.sync_copy(dst2d.at[pl.ds(base, k_per_tile)], dstv)
    pltpu.sync_copy(ones, onesv)

    iota = lax.iota(jnp.int32, LANES)
    for p in range(npass):
        lo = jnp.full((LANES,), p * hn, jnp.int32)

        def remap(r, carry):
            for kk in range(CHUNK // LANES):
                sl = pl.ds(kk * LANES, LANES)
                local = dstv[r, sl] - lo
                oob = (local < 0) | (local >= hn)
                dummy = hn + kk * LANES + iota
                dstl[r, sl] = jnp.where(oob, dummy, local)
            return carry

        lax.fori_loop(0, k_per_tile, remap, 0)
        pltpu.sync_copy(zeros, acc.at[pl.ds(s * stripe, stripe)])
        plsc.subcore_barrier()

        def step(j, carry):
            pltpu.sync_copy(onesv, acc.at[dstl.at[j]], add=True)
            return carry

        lax.fori_loop(0, k_per_tile, step, 0)
        plsc.subcore_barrier()
        pltpu.sync_copy(acc.at[pl.ds(s * stripe, stripe)],
                        out.at[c, pl.ds(p * hn + s * stripe, stripe)])


@functools.lru_cache(maxsize=None)
def _make_deg_kernel(nchunks, rpad, width, hn, npass):
    stripe = hn // NS
    k = nchunks // (NC * NS)
    return pl.kernel(
        functools.partial(_deg_body, k, stripe, hn, npass, width),
        out_type=jax.ShapeDtypeStruct((NC, rpad, width), jnp.float32),
        mesh=_mesh(),
        scratch_types=[
            pltpu.VMEM((k, CHUNK), jnp.int32),
            pltpu.VMEM((k, CHUNK), jnp.int32),
            pltpu.VMEM((CHUNK, width), jnp.float32),
            pltpu.VMEM_SHARED((hn + CHUNK, width), jnp.float32),
        ],
    )


def _scatter_body(feature_split, n, k_per_tile, stripe, hn, npass,
                  src2d, dst2d, table, zeros, out,
                  srcv, dstv, cdst, rows0, acc, sem0):
    c = lax.axis_index("c")
    s = lax.axis_index("s")
    if feature_split:
        base = s * k_per_tile
    else:
        base = (c * NS + s) * k_per_tile
    pltpu.sync_copy(src2d.at[pl.ds(base, k_per_tile)], srcv)
    pltpu.sync_copy(dst2d.at[pl.ds(base, k_per_tile)], dstv)
    if feature_split:
        off = jnp.full((LANES,), c * n, jnp.int32)

        def adj(r, carry):
            for kk in range(CHUNK // LANES):
                sl = pl.ds(kk * LANES, LANES)
                srcv[r, sl] = srcv[r, sl] + off
            return carry

        lax.fori_loop(0, k_per_tile, adj, 0)

    iota = lax.iota(jnp.int32, LANES)
    for p in range(npass):
        lo = jnp.full((LANES,), p * hn, jnp.int32)

        def remap(r, carry):
            for kk in range(CHUNK // LANES):
                sl = pl.ds(kk * LANES, LANES)
                local = dstv[r, sl] - lo
                oob = (local < 0) | (local >= hn)
                dummy = hn + kk * LANES + iota
                cdst[r, sl] = jnp.where(oob, dummy, local)
            return carry

        lax.fori_loop(0, k_per_tile, remap, 0)

        pltpu.sync_copy(zeros, acc.at[pl.ds(s * stripe, stripe)])
        plsc.subcore_barrier()

        def step(j, carry):
            pltpu.async_copy(table.at[srcv.at[j]], rows0, sem0).wait()
            pltpu.sync_copy(rows0, acc.at[cdst.at[j]], add=True)
            return carry

        lax.fori_loop(0, k_per_tile, step, 0)
        plsc.subcore_barrier()
        pltpu.sync_copy(acc.at[pl.ds(s * stripe, stripe)],
                        out.at[c, pl.ds(p * hn + s * stripe, stripe)])


@functools.lru_cache(maxsize=None)
def _make_scatter_kernel(feature_split, n, nchunks, rpad, width, hn, npass):
    stripe = hn // NS
    k = nchunks // NS if feature_split else nchunks // (NC * NS)
    return pl.kernel(
        functools.partial(_scatter_body, feature_split, n, k, stripe, hn,
                          npass),
        out_type=jax.ShapeDtypeStruct((NC, rpad, width), jnp.float32),
        mesh=_mesh(),
        scratch_types=[
            pltpu.VMEM((k, CHUNK), jnp.int32),
            pltpu.VMEM((k, CHUNK), jnp.int32),
            pltpu.VMEM((k, CHUNK), jnp.int32),
            pltpu.VMEM((CHUNK, width), jnp.float32),
            pltpu.VMEM_SHARED((hn + CHUNK, width), jnp.float32),
            pltpu.SemaphoreType.DMA,
        ],
    )


def _dis(degp_ref):
    deg = degp_ref[0, :, 0:1] + degp_ref[1, :, 0:1] + 1.0
    return lax.rsqrt(deg)


def _tc_first_body(x_ref, degp_ref, w_ref, hp_ref):
    dis = _dis(degp_ref)
    h = jnp.dot(x_ref[...], w_ref[...], preferred_element_type=jnp.float32)
    hp = h * dis
    half = hp.shape[1] // 2
    hp_ref[0] = hp[:, :half]
    hp_ref[1] = hp[:, half:]


def _tc_mid_body(split_out, degp_ref, acc_ref, hpp_ref, b_ref, w_ref, hp_ref):
    dis = _dis(degp_ref)
    cat = jnp.concatenate([acc_ref[0] + hpp_ref[0], acc_ref[1] + hpp_ref[1]],
                          axis=1)
    x = jnp.maximum(cat * dis + b_ref[...], 0.0)
    hp = jnp.dot(x, w_ref[...], preferred_element_type=jnp.float32) * dis
    if split_out:
        half = hp.shape[1] // 2
        hp_ref[0] = hp[:, :half]
        hp_ref[1] = hp[:, half:]
    else:
        hp_ref[...] = hp


def _tc_last_body(degp_ref, accp_ref, hp_ref, b_ref, out_ref):
    dis = _dis(degp_ref)
    t = (accp_ref[0] + accp_ref[1] + hp_ref[...]) * dis + b_ref[...]
    out_ref[0] = jnp.maximum(t, 0.0)


def _tc_first(x, degp, w):
    n, d_in = x.shape
    d_out = w.shape[1]
    grid = n // ROW_BLK
    return pl.pallas_call(
        _tc_first_body,
        grid=(grid,),
        in_specs=[
            pl.BlockSpec((ROW_BLK, d_in), lambda i: (i, 0)),
            pl.BlockSpec((NC, ROW_BLK, DEG_W), lambda i: (0, i, 0)),
            pl.BlockSpec((d_in, d_out), lambda i: (0, 0)),
        ],
        out_specs=pl.BlockSpec((NC, ROW_BLK, d_out // 2), lambda i: (0, i, 0)),
        out_shape=jax.ShapeDtypeStruct((NC, n, d_out // 2), jnp.float32),
    )(x, degp, w)


def _tc_mid(degp, acc, hpp, b, w, split_out):
    n = acc.shape[1]
    d_in = 2 * acc.shape[2]
    d_out = w.shape[1]
    grid = n // ROW_BLK
    if split_out:
        out_spec = pl.BlockSpec((NC, ROW_BLK, d_out // 2), lambda i: (0, i, 0))
        out_shape = jax.ShapeDtypeStruct((NC, n, d_out // 2), jnp.float32)
    else:
        out_spec = pl.BlockSpec((ROW_BLK, d_out), lambda i: (i, 0))
        out_shape = jax.ShapeDtypeStruct((n, d_out), jnp.float32)
    return pl.pallas_call(
        functools.partial(_tc_mid_body, split_out),
        grid=(grid,),
        in_specs=[
            pl.BlockSpec((NC, ROW_BLK, DEG_W), lambda i: (0, i, 0)),
            pl.BlockSpec((NC, ROW_BLK, d_in // 2), lambda i: (0, i, 0)),
            pl.BlockSpec((NC, ROW_BLK, d_in // 2), lambda i: (0, i, 0)),
            pl.BlockSpec((1, d_in), lambda i: (0, 0)),
            pl.BlockSpec((d_in, d_out), lambda i: (0, 0)),
        ],
        out_specs=out_spec,
        out_shape=out_shape,
    )(degp, acc, hpp, b, w)


def _tc_last(degp, accp, hp, b):
    n, d = hp.shape
    grid = n // ROW_BLK
    return pl.pallas_call(
        _tc_last_body,
        grid=(grid,),
        in_specs=[
            pl.BlockSpec((NC, ROW_BLK, DEG_W), lambda i: (0, i, 0)),
            pl.BlockSpec((NC, ROW_BLK, d), lambda i: (0, i, 0)),
            pl.BlockSpec((ROW_BLK, d), lambda i: (i, 0)),
            pl.BlockSpec((1, d), lambda i: (0, 0)),
        ],
        out_specs=pl.BlockSpec((1, ROW_BLK, d), lambda i: (0, i, 0)),
        out_shape=jax.ShapeDtypeStruct((1, n, d), jnp.float32),
    )(degp, accp, hp, b)


def kernel(mesh_node_features, edge_index, W1, b1, W2, b2, W3, b3):
    x = mesh_node_features[0]
    n = x.shape[0]
    e = edge_index.shape[1]
    h_dim = W1.shape[1]

    npass = 2
    hn = -(-(-(-(n + 1) // npass)) // (NS * 8)) * (NS * 8)
    rpad = npass * hn
    nchunks = -(-e // CHUNK)
    nchunks = -(-nchunks // (NC * NS * 8)) * (NC * NS * 8)
    pad_e = nchunks * CHUNK - e

    src = jnp.concatenate([edge_index[0], jnp.zeros((pad_e,), jnp.int32)])
    dst = jnp.concatenate([edge_index[1], jnp.full((pad_e,), n, jnp.int32)])
    src2d = src.reshape(nchunks, CHUNK)
    dst2d = dst.reshape(nchunks, CHUNK)

    zeros_h = jnp.zeros((hn // NS, h_dim // 2), jnp.float32)
    ones_deg = jnp.ones((CHUNK, DEG_W), jnp.float32)
    b1r = b1.reshape(1, -1)
    b2r = b2.reshape(1, -1)
    b3r = b3.reshape(1, -1)

    degp = _make_deg_kernel(nchunks, rpad, DEG_W, hn, npass)(
        dst2d, ones_deg, zeros_h)[:, :n, :]

    scat_f = _make_scatter_kernel(True, n, nchunks, rpad, h_dim // 2, hn,
                                  npass)
    scat_e = _make_scatter_kernel(False, n, nchunks, rpad, h_dim // 2, hn,
                                  npass)

    hp1 = _tc_first(x, degp, W1)
    acc1 = scat_f(src2d, dst2d, hp1.reshape(NC * n, -1), zeros_h)[:, :n, :]
    hp2 = _tc_mid(degp, acc1, hp1, b1r, W2, split_out=True)
    acc2 = scat_f(src2d, dst2d, hp2.reshape(NC * n, -1), zeros_h)[:, :n, :]
    hp3 = _tc_mid(degp, acc2, hp2, b2r, W3, split_out=False)
    acc3 = scat_e(src2d, dst2d, hp3, zeros_h)[:, :n, :]
    return _tc_last(degp, acc3, hp3, b3r)
